# baseline (speedup 1.0000x reference)
import jax
import jax.numpy as jnp
from jax.experimental import pallas as pl
from jax.experimental.pallas import tpu as pltpu


def _fused_mlp_body(x_ref, w1t_ref, b1_ref, w2t_ref, b2_ref, o_ref):
    h = jax.lax.dot_general(
        x_ref[...], w1t_ref[...],
        dimension_numbers=(((1,), (0,)), ((), ())),
        preferred_element_type=jnp.float32,
    )
    h = jnp.maximum(h + b1_ref[...], 0.0)
    out = jax.lax.dot_general(
        h, w2t_ref[...],
        dimension_numbers=(((1,), (0,)), ((), ())),
        preferred_element_type=jnp.float32,
    )
    o_ref[...] = out + b2_ref[...]


def kernel(x, w1, b1, w2, b2):
    B, latent = x.shape
    H = w1.shape[0]
    C = w2.shape[0]

    w1t = jnp.transpose(w1)
    w2t = jnp.transpose(w2)
    b1r = b1.reshape(1, H)
    b2r = b2.reshape(1, C)

    tm = 8192
    grid = (pl.cdiv(B, tm),)

    return pl.pallas_call(
        _fused_mlp_body,
        out_shape=jax.ShapeDtypeStruct((B, C), x.dtype),
        grid=grid,
        in_specs=[
            pl.BlockSpec((tm, latent), lambda i: (i, 0)),
            pl.BlockSpec((latent, H), lambda i: (0, 0)),
            pl.BlockSpec((1, H), lambda i: (0, 0)),
            pl.BlockSpec((H, C), lambda i: (0, 0)),
            pl.BlockSpec((1, C), lambda i: (0, 0)),
        ],
        out_specs=pl.BlockSpec((tm, C), lambda i: (i, 0)),
        compiler_params=pltpu.CompilerParams(
            dimension_semantics=("parallel",),
            vmem_limit_bytes=64 * 1024 * 1024,
        ),
    )(x, w1t, b1r, w2t, b2r)
